# speedup vs baseline: 1.3319x; 1.3319x over previous
import jax
import jax.numpy as jnp
from jax.experimental import pallas as pl
from jax.experimental.pallas import tpu as pltpu

_DIM = 1024
_SBLK = 512


def _body(keep_ref, x_ref, tok_ref, o_ref):
    j = pl.program_id(1)
    keep = keep_ref[0]
    pos = j * _SBLK + jax.lax.broadcasted_iota(jnp.int32, (1, _SBLK, _DIM), 1)
    tok = tok_ref[...][:, None, :]
    o_ref[...] = jnp.where(pos >= keep, tok, x_ref[...])


def kernel(x, mask_token, keep_k):
    B, N, D = x.shape
    keep_arr = jnp.atleast_1d(jnp.asarray(keep_k, jnp.int32))
    tok2d = mask_token.reshape(1, D)

    def x_map(i, j, keep_ref):
        last_kept = jnp.maximum(pl.cdiv(keep_ref[0], _SBLK) - 1, 0)
        return (i, jnp.minimum(j, last_kept), 0)

    grid_spec = pltpu.PrefetchScalarGridSpec(
        num_scalar_prefetch=1,
        grid=(B, N // _SBLK),
        in_specs=[
            pl.BlockSpec((1, _SBLK, D), x_map),
            pl.BlockSpec((1, D), lambda i, j, k: (0, 0)),
        ],
        out_specs=pl.BlockSpec((1, _SBLK, D), lambda i, j, k: (i, j, 0)),
    )
    return pl.pallas_call(
        _body,
        grid_spec=grid_spec,
        out_shape=jax.ShapeDtypeStruct((B, N, D), x.dtype),
        compiler_params=pltpu.CompilerParams(
            dimension_semantics=("arbitrary", "arbitrary"),
        ),
    )(keep_arr, x, tok2d)
